# Initial kernel scaffold; baseline (speedup 1.0000x reference)
import jax
import jax.numpy as jnp
from jax import lax
from jax.experimental import pallas as pl
from jax.experimental.pallas import tpu as pltpu
from jax.experimental.pallas import tpu_sc as plsc

N = 10000
D = 128
E = 320000

NC = 2
NS = 16
NW = NC * NS
CH = 128
K = 80
EPT = K * CH
E_PAD = NW * EPT

ACC_ROWS = 10240
ZR = ACC_ROWS // NS
DR = N // NS

_MESH = plsc.VectorSubcoreMesh(core_axis_name="c", subcore_axis_name="s")


def _deg_body(colt_hbm, zeros_hbm, ones_hbm, out_hbm, coli, ones_v, acc):
    c = lax.axis_index("c")
    s = lax.axis_index("s")
    wid = c * NS + s
    pltpu.sync_copy(zeros_hbm, acc.at[pl.ds(s * ZR, ZR)])
    pltpu.sync_copy(colt_hbm.at[wid], coli)
    pltpu.sync_copy(ones_hbm, ones_v)
    plsc.subcore_barrier()

    @pl.loop(0, K)
    def _(j):
        pltpu.sync_copy(ones_v, acc.at[coli.at[j]], add=True)

    plsc.subcore_barrier()
    pltpu.sync_copy(acc.at[pl.ds(s * DR, DR)], out_hbm.at[c, pl.ds(s * DR, DR)])


_sc_deg = pl.kernel(
    _deg_body,
    out_type=jax.ShapeDtypeStruct((NC, N, 16), jnp.float32),
    mesh=_MESH,
    scratch_types=[
        pltpu.VMEM((K, CH), jnp.int32),
        pltpu.VMEM((CH, 16), jnp.float32),
        pltpu.VMEM_SHARED((ACC_ROWS, 16), jnp.float32),
    ],
)


def _prop_body(h_hbm, rowt_hbm, colt_hbm, zeros_hbm, out_hbm,
               rowi, coli, rows, acc, sem):
    c = lax.axis_index("c")
    s = lax.axis_index("s")
    wid = c * NS + s
    pltpu.sync_copy(zeros_hbm, acc.at[pl.ds(s * ZR, ZR)])
    pltpu.sync_copy(rowt_hbm.at[wid], rowi)
    pltpu.sync_copy(colt_hbm.at[wid], coli)
    plsc.subcore_barrier()

    @pl.loop(0, K)
    def _(j):
        pltpu.async_copy(h_hbm.at[rowi.at[j]], rows, sem).wait()
        pltpu.sync_copy(rows, acc.at[coli.at[j]], add=True)

    plsc.subcore_barrier()
    pltpu.sync_copy(acc.at[pl.ds(s * DR, DR)], out_hbm.at[c, pl.ds(s * DR, DR)])


_sc_prop = pl.kernel(
    _prop_body,
    out_type=jax.ShapeDtypeStruct((NC, N, D), jnp.float32),
    mesh=_MESH,
    scratch_types=[
        pltpu.VMEM((K, CH), jnp.int32),
        pltpu.VMEM((K, CH), jnp.int32),
        pltpu.VMEM((CH, D), jnp.float32),
        pltpu.VMEM_SHARED((ACC_ROWS, D), jnp.float32),
        pltpu.SemaphoreType.DMA,
    ],
)

_BLK = 1000
_GRID = N // _BLK
_EPS = 1e-12


def _tc_a_body(x_ref, w_ref, b_ref, degp_ref, h1_ref, dinv_ref):
    xb = x_ref[...]
    nrm = jnp.sqrt(jnp.sum(xb * xb, axis=1, keepdims=True))
    xn = xb / jnp.maximum(nrm, _EPS)
    z = jnp.dot(xn, w_ref[...], preferred_element_type=jnp.float32) + b_ref[...]
    deg = degp_ref[0] + degp_ref[1] + 1.0
    dinv = lax.rsqrt(deg)
    dinv_ref[...] = dinv
    h1_ref[...] = z * dinv[:, 0:1]


_tc_a = pl.pallas_call(
    _tc_a_body,
    grid=(_GRID,),
    in_specs=[
        pl.BlockSpec((_BLK, D), lambda i: (i, 0)),
        pl.BlockSpec((D, D), lambda i: (0, 0)),
        pl.BlockSpec((1, D), lambda i: (0, 0)),
        pl.BlockSpec((NC, _BLK, 16), lambda i: (0, i, 0)),
    ],
    out_specs=[
        pl.BlockSpec((_BLK, D), lambda i: (i, 0)),
        pl.BlockSpec((_BLK, 16), lambda i: (i, 0)),
    ],
    out_shape=[
        jax.ShapeDtypeStruct((N, D), jnp.float32),
        jax.ShapeDtypeStruct((N, 16), jnp.float32),
    ],
)


def _tc_b_body(aggp_ref, h1_ref, dinv_ref, wmu_ref, bmu_ref, wvar_ref,
               bvar_ref, hmu_ref, hvar_ref):
    dinv = dinv_ref[...][:, 0:1]
    hid = jnp.maximum((aggp_ref[0] + aggp_ref[1] + h1_ref[...]) * dinv, 0.0)
    zmu = jnp.dot(hid, wmu_ref[...], preferred_element_type=jnp.float32) + bmu_ref[...]
    nmu = jnp.sqrt(jnp.sum(zmu * zmu, axis=1, keepdims=True))
    hmu_ref[...] = zmu / jnp.maximum(nmu, _EPS) * dinv
    zva = jnp.dot(hid, wvar_ref[...], preferred_element_type=jnp.float32) + bvar_ref[...]
    nva = jnp.sqrt(jnp.sum(zva * zva, axis=1, keepdims=True))
    hvar_ref[...] = zva / jnp.maximum(nva, _EPS) * dinv


_tc_b = pl.pallas_call(
    _tc_b_body,
    grid=(_GRID,),
    in_specs=[
        pl.BlockSpec((NC, _BLK, D), lambda i: (0, i, 0)),
        pl.BlockSpec((_BLK, D), lambda i: (i, 0)),
        pl.BlockSpec((_BLK, 16), lambda i: (i, 0)),
        pl.BlockSpec((D, D), lambda i: (0, 0)),
        pl.BlockSpec((1, D), lambda i: (0, 0)),
        pl.BlockSpec((D, D), lambda i: (0, 0)),
        pl.BlockSpec((1, D), lambda i: (0, 0)),
    ],
    out_specs=[
        pl.BlockSpec((_BLK, D), lambda i: (i, 0)),
        pl.BlockSpec((_BLK, D), lambda i: (i, 0)),
    ],
    out_shape=[
        jax.ShapeDtypeStruct((N, D), jnp.float32),
        jax.ShapeDtypeStruct((N, D), jnp.float32),
    ],
)


def _tc_c_body(amup_ref, avap_ref, hmu_ref, hvar_ref, dinv_ref,
               mu_ref, var_ref):
    dinv = dinv_ref[...][:, 0:1]
    mu_ref[...] = (amup_ref[0] + amup_ref[1] + hmu_ref[...]) * dinv
    var_ref[...] = (avap_ref[0] + avap_ref[1] + hvar_ref[...]) * dinv


_tc_c = pl.pallas_call(
    _tc_c_body,
    grid=(_GRID,),
    in_specs=[
        pl.BlockSpec((NC, _BLK, D), lambda i: (0, i, 0)),
        pl.BlockSpec((NC, _BLK, D), lambda i: (0, i, 0)),
        pl.BlockSpec((_BLK, D), lambda i: (i, 0)),
        pl.BlockSpec((_BLK, D), lambda i: (i, 0)),
        pl.BlockSpec((_BLK, 16), lambda i: (i, 0)),
    ],
    out_specs=[
        pl.BlockSpec((_BLK, D), lambda i: (i, 0)),
        pl.BlockSpec((_BLK, D), lambda i: (i, 0)),
    ],
    out_shape=[
        jax.ShapeDtypeStruct((N, D), jnp.float32),
        jax.ShapeDtypeStruct((N, D), jnp.float32),
    ],
)


def kernel(x, edge_index, W0, b0, W_mu, b_mu, W_var, b_var):
    ei = edge_index.astype(jnp.int32)
    pad = E_PAD - E
    rowp = jnp.concatenate([ei[0], jnp.zeros((pad,), jnp.int32)])
    colp = jnp.concatenate([ei[1], jnp.full((pad,), N, jnp.int32)])
    rowt = rowp.reshape(NW, K, CH)
    colt = colp.reshape(NW, K, CH)

    zeros16 = jnp.zeros((ZR, 16), jnp.float32)
    zerosD = jnp.zeros((ZR, D), jnp.float32)
    ones16 = jnp.ones((CH, 16), jnp.float32)

    degp = _sc_deg(colt, zeros16, ones16)
    h1, dinv = _tc_a(x, W0, b0.reshape(1, D), degp)
    aggp = _sc_prop(h1, rowt, colt, zerosD)
    hmu, hvar = _tc_b(aggp, h1, dinv, W_mu, b_mu.reshape(1, D),
                      W_var, b_var.reshape(1, D))
    amup = _sc_prop(hmu, rowt, colt, zerosD)
    avap = _sc_prop(hvar, rowt, colt, zerosD)
    mu, var = _tc_c(amup, avap, hmu, hvar, dinv)
    return (mu, var)


# speedup vs baseline: 6.7970x; 6.7970x over previous
import jax
import jax.numpy as jnp
from jax import lax
from jax.experimental import pallas as pl
from jax.experimental.pallas import tpu as pltpu
from jax.experimental.pallas import tpu_sc as plsc

N = 10000
D = 128
E = 320000

NC = 2
NS = 16
NW = NC * NS
CH = 128
K = 80
EPT = K * CH
E_PAD = NW * EPT

ACC_ROWS = 10240
ZR = ACC_ROWS // NS

_MESH = plsc.VectorSubcoreMesh(core_axis_name="c", subcore_axis_name="s")


def _deg_body(colt_hbm, zeros_hbm, ones_hbm, out_hbm, coli, ones_v, acc):
    c = lax.axis_index("c")
    s = lax.axis_index("s")
    wid = c * NS + s
    pltpu.sync_copy(zeros_hbm, acc.at[pl.ds(s * ZR, ZR)])
    pltpu.sync_copy(colt_hbm.at[wid], coli)
    pltpu.sync_copy(ones_hbm, ones_v)
    plsc.subcore_barrier()

    @pl.loop(0, K)
    def _(j):
        pltpu.sync_copy(ones_v, acc.at[coli.at[j]], add=True)

    plsc.subcore_barrier()
    pltpu.sync_copy(acc.at[pl.ds(s * ZR, ZR)], out_hbm.at[c, pl.ds(s * ZR, ZR)])


_sc_deg = pl.kernel(
    _deg_body,
    out_type=jax.ShapeDtypeStruct((NC, ACC_ROWS, D), jnp.float32),
    mesh=_MESH,
    scratch_types=[
        pltpu.VMEM((K, CH), jnp.int32),
        pltpu.VMEM((CH, D), jnp.float32),
        pltpu.VMEM_SHARED((ACC_ROWS, D), jnp.float32),
    ],
)


def _prop_body(h_hbm, rowt_hbm, colt_hbm, zeros_hbm, out_hbm,
               rowi, coli, rows, acc, sem):
    c = lax.axis_index("c")
    s = lax.axis_index("s")
    wid = c * NS + s
    pltpu.sync_copy(zeros_hbm, acc.at[pl.ds(s * ZR, ZR)])
    pltpu.sync_copy(rowt_hbm.at[wid], rowi)
    pltpu.sync_copy(colt_hbm.at[wid], coli)
    plsc.subcore_barrier()

    @pl.loop(0, K)
    def _(j):
        pltpu.async_copy(h_hbm.at[rowi.at[j]], rows, sem).wait()
        pltpu.sync_copy(rows, acc.at[coli.at[j]], add=True)

    plsc.subcore_barrier()
    pltpu.sync_copy(acc.at[pl.ds(s * ZR, ZR)], out_hbm.at[c, pl.ds(s * ZR, ZR)])


_sc_prop = pl.kernel(
    _prop_body,
    out_type=jax.ShapeDtypeStruct((NC, ACC_ROWS, D), jnp.float32),
    mesh=_MESH,
    scratch_types=[
        pltpu.VMEM((K, CH), jnp.int32),
        pltpu.VMEM((K, CH), jnp.int32),
        pltpu.VMEM((CH, D), jnp.float32),
        pltpu.VMEM_SHARED((ACC_ROWS, D), jnp.float32),
        pltpu.SemaphoreType.DMA,
    ],
)

_BLK = 1000
_GRID = N // _BLK
_EPS = 1e-12


def _tc_a_body(x_ref, w_ref, b_ref, degp_ref, h1_ref, dinv_ref):
    xb = x_ref[...]
    nrm = jnp.sqrt(jnp.sum(xb * xb, axis=1, keepdims=True))
    xn = xb / jnp.maximum(nrm, _EPS)
    z = jnp.dot(xn, w_ref[...], preferred_element_type=jnp.float32) + b_ref[...]
    deg = degp_ref[0] + degp_ref[1] + 1.0
    dinv = lax.rsqrt(deg)
    dinv_ref[...] = dinv[:, 0:16]
    h1_ref[...] = z * dinv


_tc_a = pl.pallas_call(
    _tc_a_body,
    grid=(_GRID,),
    in_specs=[
        pl.BlockSpec((_BLK, D), lambda i: (i, 0)),
        pl.BlockSpec((D, D), lambda i: (0, 0)),
        pl.BlockSpec((1, D), lambda i: (0, 0)),
        pl.BlockSpec((NC, _BLK, D), lambda i: (0, i, 0)),
    ],
    out_specs=[
        pl.BlockSpec((_BLK, D), lambda i: (i, 0)),
        pl.BlockSpec((_BLK, 16), lambda i: (i, 0)),
    ],
    out_shape=[
        jax.ShapeDtypeStruct((N, D), jnp.float32),
        jax.ShapeDtypeStruct((N, 16), jnp.float32),
    ],
)


def _tc_b_body(aggp_ref, h1_ref, dinv_ref, wmu_ref, bmu_ref, wvar_ref,
               bvar_ref, hmu_ref, hvar_ref):
    dinv = dinv_ref[...][:, 0:1]
    hid = jnp.maximum((aggp_ref[0] + aggp_ref[1] + h1_ref[...]) * dinv, 0.0)
    zmu = jnp.dot(hid, wmu_ref[...], preferred_element_type=jnp.float32) + bmu_ref[...]
    nmu = jnp.sqrt(jnp.sum(zmu * zmu, axis=1, keepdims=True))
    hmu_ref[...] = zmu / jnp.maximum(nmu, _EPS) * dinv
    zva = jnp.dot(hid, wvar_ref[...], preferred_element_type=jnp.float32) + bvar_ref[...]
    nva = jnp.sqrt(jnp.sum(zva * zva, axis=1, keepdims=True))
    hvar_ref[...] = zva / jnp.maximum(nva, _EPS) * dinv


_tc_b = pl.pallas_call(
    _tc_b_body,
    grid=(_GRID,),
    in_specs=[
        pl.BlockSpec((NC, _BLK, D), lambda i: (0, i, 0)),
        pl.BlockSpec((_BLK, D), lambda i: (i, 0)),
        pl.BlockSpec((_BLK, 16), lambda i: (i, 0)),
        pl.BlockSpec((D, D), lambda i: (0, 0)),
        pl.BlockSpec((1, D), lambda i: (0, 0)),
        pl.BlockSpec((D, D), lambda i: (0, 0)),
        pl.BlockSpec((1, D), lambda i: (0, 0)),
    ],
    out_specs=[
        pl.BlockSpec((_BLK, D), lambda i: (i, 0)),
        pl.BlockSpec((_BLK, D), lambda i: (i, 0)),
    ],
    out_shape=[
        jax.ShapeDtypeStruct((N, D), jnp.float32),
        jax.ShapeDtypeStruct((N, D), jnp.float32),
    ],
)


def _tc_c_body(amup_ref, avap_ref, hmu_ref, hvar_ref, dinv_ref,
               mu_ref, var_ref):
    dinv = dinv_ref[...][:, 0:1]
    mu_ref[...] = (amup_ref[0] + amup_ref[1] + hmu_ref[...]) * dinv
    var_ref[...] = (avap_ref[0] + avap_ref[1] + hvar_ref[...]) * dinv


_tc_c = pl.pallas_call(
    _tc_c_body,
    grid=(_GRID,),
    in_specs=[
        pl.BlockSpec((NC, _BLK, D), lambda i: (0, i, 0)),
        pl.BlockSpec((NC, _BLK, D), lambda i: (0, i, 0)),
        pl.BlockSpec((_BLK, D), lambda i: (i, 0)),
        pl.BlockSpec((_BLK, D), lambda i: (i, 0)),
        pl.BlockSpec((_BLK, 16), lambda i: (i, 0)),
    ],
    out_specs=[
        pl.BlockSpec((_BLK, D), lambda i: (i, 0)),
        pl.BlockSpec((_BLK, D), lambda i: (i, 0)),
    ],
    out_shape=[
        jax.ShapeDtypeStruct((N, D), jnp.float32),
        jax.ShapeDtypeStruct((N, D), jnp.float32),
    ],
)


def kernel(x, edge_index, W0, b0, W_mu, b_mu, W_var, b_var):
    ei = edge_index.astype(jnp.int32)
    pad = E_PAD - E
    rowp = jnp.concatenate([ei[0], jnp.zeros((pad,), jnp.int32)])
    colp = jnp.concatenate([ei[1], jnp.full((pad,), N, jnp.int32)])
    rowt = rowp.reshape(NW, K, CH)
    colt = colp.reshape(NW, K, CH)

    zerosD = jnp.zeros((ZR, D), jnp.float32)
    onesD = jnp.ones((CH, D), jnp.float32)

    degp = _sc_deg(colt, zerosD, onesD)
    h1, dinv = _tc_a(x, W0, b0.reshape(1, D), degp)
    aggp = _sc_prop(h1, rowt, colt, zerosD)
    hmu, hvar = _tc_b(aggp, h1, dinv, W_mu, b_mu.reshape(1, D),
                      W_var, b_var.reshape(1, D))
    amup = _sc_prop(hmu, rowt, colt, zerosD)
    avap = _sc_prop(hvar, rowt, colt, zerosD)
    mu, var = _tc_c(amup, avap, hmu, hvar, dinv)
    return (mu, var)


# speedup vs baseline: 7.0740x; 1.0408x over previous
import jax
import jax.numpy as jnp
from jax import lax
from jax.experimental import pallas as pl
from jax.experimental.pallas import tpu as pltpu
from jax.experimental.pallas import tpu_sc as plsc

N = 10000
D = 128
E = 320000

NC = 2
NS = 16
NW = NC * NS
CH = 128
PH = 2
KP = 40
K = PH * KP
EPT = K * CH
E_PAD = NW * EPT

ACC_ROWS = 10112
ZR = ACC_ROWS // NS

_MESH = plsc.VectorSubcoreMesh(core_axis_name="c", subcore_axis_name="s")


def _deg_body(colt_hbm, zeros_hbm, ones_hbm, out_hbm, coli, ones_v, acc):
    c = lax.axis_index("c")
    s = lax.axis_index("s")
    wid = c * NS + s
    pltpu.sync_copy(zeros_hbm, acc.at[pl.ds(s * ZR, ZR)])
    pltpu.sync_copy(colt_hbm.at[wid], coli)
    pltpu.sync_copy(ones_hbm, ones_v)
    plsc.subcore_barrier()

    @pl.loop(0, K)
    def _(j):
        pltpu.sync_copy(ones_v, acc.at[coli.at[j]], add=True)

    plsc.subcore_barrier()
    pltpu.sync_copy(acc.at[pl.ds(s * ZR, ZR)], out_hbm.at[c, pl.ds(s * ZR, ZR)])


_sc_deg = pl.kernel(
    _deg_body,
    out_type=jax.ShapeDtypeStruct((NC, ACC_ROWS, D), jnp.float32),
    mesh=_MESH,
    scratch_types=[
        pltpu.VMEM((K, CH), jnp.int32),
        pltpu.VMEM((CH, D), jnp.float32),
        pltpu.VMEM_SHARED((ACC_ROWS, D), jnp.float32),
    ],
)


NBUF = 2


def _prop_body(h_hbm, rowt_hbm, colt_hbm, zeros_hbm, out_hbm,
               rowi, coli, rows, acc, semg):
    c = lax.axis_index("c")
    s = lax.axis_index("s")
    wid = c * NS + s
    pltpu.sync_copy(zeros_hbm, acc.at[pl.ds(s * ZR, ZR)])
    plsc.subcore_barrier()

    for ph in range(PH):
        pltpu.sync_copy(rowt_hbm.at[wid, pl.ds(ph * KP, KP)], rowi)
        pltpu.sync_copy(colt_hbm.at[wid, pl.ds(ph * KP, KP)], coli)

        for j in range(NBUF - 1):
            pltpu.async_copy(h_hbm.at[rowi.at[j]], rows.at[j], semg)

        @pl.loop(0, KP)
        def _(j):
            p = lax.rem(j, NBUF)
            pltpu.make_async_copy(h_hbm.at[rowi.at[j]], rows.at[p], semg).wait()

            @pl.when(j + NBUF - 1 < KP)
            def _():
                pltpu.async_copy(h_hbm.at[rowi.at[j + NBUF - 1]],
                                 rows.at[lax.rem(j + NBUF - 1, NBUF)], semg)

            pltpu.sync_copy(rows.at[p], acc.at[coli.at[j]], add=True)

    plsc.subcore_barrier()
    pltpu.sync_copy(acc.at[pl.ds(s * ZR, ZR)], out_hbm.at[c, pl.ds(s * ZR, ZR)])


_sc_prop = pl.kernel(
    _prop_body,
    out_type=jax.ShapeDtypeStruct((NC, ACC_ROWS, D), jnp.float32),
    mesh=_MESH,
    scratch_types=[
        pltpu.VMEM((KP, CH), jnp.int32),
        pltpu.VMEM((KP, CH), jnp.int32),
        pltpu.VMEM((NBUF, CH, D), jnp.float32),
        pltpu.VMEM_SHARED((ACC_ROWS, D), jnp.float32),
        pltpu.SemaphoreType.DMA,
    ],
)

_BLK = 1000
_GRID = N // _BLK
_EPS = 1e-12


def _tc_a_body(x_ref, w_ref, b_ref, degp_ref, h1_ref, dinv_ref):
    xb = x_ref[...]
    nrm = jnp.sqrt(jnp.sum(xb * xb, axis=1, keepdims=True))
    xn = xb / jnp.maximum(nrm, _EPS)
    z = jnp.dot(xn, w_ref[...], preferred_element_type=jnp.float32) + b_ref[...]
    deg = degp_ref[0] + degp_ref[1] + 1.0
    dinv = lax.rsqrt(deg)
    dinv_ref[...] = dinv[:, 0:16]
    h1_ref[...] = z * dinv


_tc_a = pl.pallas_call(
    _tc_a_body,
    grid=(_GRID,),
    in_specs=[
        pl.BlockSpec((_BLK, D), lambda i: (i, 0)),
        pl.BlockSpec((D, D), lambda i: (0, 0)),
        pl.BlockSpec((1, D), lambda i: (0, 0)),
        pl.BlockSpec((NC, _BLK, D), lambda i: (0, i, 0)),
    ],
    out_specs=[
        pl.BlockSpec((_BLK, D), lambda i: (i, 0)),
        pl.BlockSpec((_BLK, 16), lambda i: (i, 0)),
    ],
    out_shape=[
        jax.ShapeDtypeStruct((N, D), jnp.float32),
        jax.ShapeDtypeStruct((N, 16), jnp.float32),
    ],
)


def _tc_b_body(aggp_ref, h1_ref, dinv_ref, wmu_ref, bmu_ref, wvar_ref,
               bvar_ref, hmu_ref, hvar_ref):
    dinv = dinv_ref[...][:, 0:1]
    hid = jnp.maximum((aggp_ref[0] + aggp_ref[1] + h1_ref[...]) * dinv, 0.0)
    zmu = jnp.dot(hid, wmu_ref[...], preferred_element_type=jnp.float32) + bmu_ref[...]
    nmu = jnp.sqrt(jnp.sum(zmu * zmu, axis=1, keepdims=True))
    hmu_ref[...] = zmu / jnp.maximum(nmu, _EPS) * dinv
    zva = jnp.dot(hid, wvar_ref[...], preferred_element_type=jnp.float32) + bvar_ref[...]
    nva = jnp.sqrt(jnp.sum(zva * zva, axis=1, keepdims=True))
    hvar_ref[...] = zva / jnp.maximum(nva, _EPS) * dinv


_tc_b = pl.pallas_call(
    _tc_b_body,
    grid=(_GRID,),
    in_specs=[
        pl.BlockSpec((NC, _BLK, D), lambda i: (0, i, 0)),
        pl.BlockSpec((_BLK, D), lambda i: (i, 0)),
        pl.BlockSpec((_BLK, 16), lambda i: (i, 0)),
        pl.BlockSpec((D, D), lambda i: (0, 0)),
        pl.BlockSpec((1, D), lambda i: (0, 0)),
        pl.BlockSpec((D, D), lambda i: (0, 0)),
        pl.BlockSpec((1, D), lambda i: (0, 0)),
    ],
    out_specs=[
        pl.BlockSpec((_BLK, D), lambda i: (i, 0)),
        pl.BlockSpec((_BLK, D), lambda i: (i, 0)),
    ],
    out_shape=[
        jax.ShapeDtypeStruct((N, D), jnp.float32),
        jax.ShapeDtypeStruct((N, D), jnp.float32),
    ],
)


def _tc_c_body(amup_ref, avap_ref, hmu_ref, hvar_ref, dinv_ref,
               mu_ref, var_ref):
    dinv = dinv_ref[...][:, 0:1]
    mu_ref[...] = (amup_ref[0] + amup_ref[1] + hmu_ref[...]) * dinv
    var_ref[...] = (avap_ref[0] + avap_ref[1] + hvar_ref[...]) * dinv


_tc_c = pl.pallas_call(
    _tc_c_body,
    grid=(_GRID,),
    in_specs=[
        pl.BlockSpec((NC, _BLK, D), lambda i: (0, i, 0)),
        pl.BlockSpec((NC, _BLK, D), lambda i: (0, i, 0)),
        pl.BlockSpec((_BLK, D), lambda i: (i, 0)),
        pl.BlockSpec((_BLK, D), lambda i: (i, 0)),
        pl.BlockSpec((_BLK, 16), lambda i: (i, 0)),
    ],
    out_specs=[
        pl.BlockSpec((_BLK, D), lambda i: (i, 0)),
        pl.BlockSpec((_BLK, D), lambda i: (i, 0)),
    ],
    out_shape=[
        jax.ShapeDtypeStruct((N, D), jnp.float32),
        jax.ShapeDtypeStruct((N, D), jnp.float32),
    ],
)


def kernel(x, edge_index, W0, b0, W_mu, b_mu, W_var, b_var):
    ei = edge_index.astype(jnp.int32)
    pad = E_PAD - E
    rowp = jnp.concatenate([ei[0], jnp.zeros((pad,), jnp.int32)])
    colp = jnp.concatenate([ei[1], jnp.full((pad,), N, jnp.int32)])
    rowt = rowp.reshape(NW, K, CH)
    colt = colp.reshape(NW, K, CH)

    zerosD = jnp.zeros((ZR, D), jnp.float32)
    onesD = jnp.ones((CH, D), jnp.float32)

    degp = _sc_deg(colt, zerosD, onesD)
    h1, dinv = _tc_a(x, W0, b0.reshape(1, D), degp)
    aggp = _sc_prop(h1, rowt, colt, zerosD)
    hmu, hvar = _tc_b(aggp, h1, dinv, W_mu, b_mu.reshape(1, D),
                      W_var, b_var.reshape(1, D))
    amup = _sc_prop(hmu, rowt, colt, zerosD)
    avap = _sc_prop(hvar, rowt, colt, zerosD)
    mu, var = _tc_c(amup, avap, hmu, hvar, dinv)
    return (mu, var)
